# baseline (speedup 1.0000x reference)
import functools

import jax
import jax.numpy as jnp
from jax.experimental import pallas as pl
from jax.experimental.pallas import tpu as pltpu

_OFFSET = 16384
_W = 128
_BG = 4096


def _body(x_ref, ox_ref):
    ox_ref[...] = x_ref[...] + x_ref.dtype.type(_OFFSET)


@functools.lru_cache(maxsize=None)
def _make_call(g: int, idx_dtype: str):
    idt = jnp.dtype(idx_dtype)
    grid = -(-g // _BG)
    return pl.pallas_call(
        _body,
        grid=(grid,),
        in_specs=[pl.BlockSpec((_BG, _W), lambda i: (i, 0))],
        out_specs=pl.BlockSpec((_BG, _W), lambda i: (i, 0)),
        out_shape=jax.ShapeDtypeStruct((g, _W), idt),
        compiler_params=pltpu.CompilerParams(
            dimension_semantics=("arbitrary",),
        ),
    )


def kernel(indices, values):
    nnz, ncols = indices.shape
    total = nnz * ncols
    g = -(-total // _W)
    pad = g * _W - total
    flat = indices.reshape(total)
    x2 = jnp.pad(flat, (0, pad)).reshape(g, _W)
    y2 = _make_call(g, str(indices.dtype))(x2)
    out = y2.reshape(g * _W)[:total].reshape(nnz, ncols)
    return (out, values)


# speedup vs baseline: 1.0023x; 1.0023x over previous
import functools

import jax
import jax.numpy as jnp
from jax.experimental import pallas as pl
from jax.experimental.pallas import tpu as pltpu

_OFFSET = 16384
_W = 128
_BG = 4096


def _body(x_ref, ox_ref):
    ox_ref[...] = x_ref[...] + x_ref.dtype.type(_OFFSET - 2)


@functools.lru_cache(maxsize=None)
def _make_call(g: int, idx_dtype: str):
    idt = jnp.dtype(idx_dtype)
    grid = -(-g // _BG)
    return pl.pallas_call(
        _body,
        grid=(grid,),
        in_specs=[pl.BlockSpec((_BG, _W), lambda i: (i, 0))],
        out_specs=pl.BlockSpec((_BG, _W), lambda i: (i, 0)),
        out_shape=jax.ShapeDtypeStruct((g, _W), idt),
        compiler_params=pltpu.CompilerParams(
            dimension_semantics=("arbitrary",),
        ),
    )


def kernel(indices, values):
    nnz, ncols = indices.shape
    total = nnz * ncols
    g = -(-total // _W)
    pad = g * _W - total
    flat = indices.reshape(total)
    x2 = jnp.pad(flat, (0, pad)).reshape(g, _W) + jnp.int32(1)
    y2 = _make_call(g, str(indices.dtype))(x2)
    out = y2.reshape(g * _W)[:total].reshape(nnz, ncols) + jnp.int32(1)
    return (out, values)


# speedup vs baseline: 290.8944x; 290.2169x over previous
import functools

import jax
import jax.numpy as jnp
from jax.experimental import pallas as pl
from jax.experimental.pallas import tpu as pltpu

_OFFSET = 16384
_BC = 1048576


def _body(x_ref, ox_ref):
    ox_ref[...] = x_ref[...] + x_ref.dtype.type(_OFFSET)


@functools.lru_cache(maxsize=None)
def _make_call(nrows: int, nnz: int, idx_dtype: str):
    idt = jnp.dtype(idx_dtype)
    grid = -(-nnz // _BC)
    return pl.pallas_call(
        _body,
        grid=(grid,),
        in_specs=[pl.BlockSpec((nrows, _BC), lambda i: (0, i))],
        out_specs=pl.BlockSpec((nrows, _BC), lambda i: (0, i)),
        out_shape=jax.ShapeDtypeStruct((nrows, nnz), idt),
        compiler_params=pltpu.CompilerParams(
            dimension_semantics=("arbitrary",),
        ),
    )


def kernel(indices, values):
    nnz, ncols = indices.shape
    xt = indices.T
    yt = _make_call(ncols, nnz, str(indices.dtype))(xt)
    return (yt.T, values)
